# Initial kernel scaffold; baseline (speedup 1.0000x reference)
import functools

import jax
import jax.numpy as jnp
from jax import lax
from jax.experimental import pallas as pl
from jax.experimental.pallas import tpu as pltpu
from jax.experimental.pallas import tpu_sc as plsc

_NUM_ITEMS = 1000000
_BATCH = 16384
_HIST = 200
_N = _BATCH * _HIST

_PAD_ROWS = 8192
_PAD_N = _PAD_ROWS * 128

_info = plsc.get_sparse_core_info()
_NC, _NS = _info.num_cores, _info.num_subcores
_NW = _NC * _NS
_PER_W = _N // _NW
_CHUNK = 51200
_NCHUNKS = _PER_W // _CHUNK


def _sigmoid_body(u_ref, x_ref, o_ref):
    o_ref[...] = jax.nn.sigmoid(u_ref[0] - x_ref[...])


def _sigmoid_table(user_params, table_pad):
    return pl.pallas_call(
        _sigmoid_body,
        out_shape=jax.ShapeDtypeStruct((_PAD_ROWS, 128), jnp.float32),
        in_specs=[
            pl.BlockSpec(memory_space=pltpu.SMEM),
            pl.BlockSpec(memory_space=pltpu.ANY),
        ],
        out_specs=pl.BlockSpec(memory_space=pltpu.ANY),
    )(user_params, table_pad)


def _gather_body(tbl_hbm, idx_hbm, out_hbm, idx_v, rows_v, sem):
    wid = lax.axis_index("s") * _NC + lax.axis_index("c")
    base = wid * _PER_W
    for k in range(_NCHUNKS):
        off = base + k * _CHUNK
        pltpu.sync_copy(idx_hbm.at[pl.ds(off, _CHUNK)], idx_v)
        pltpu.async_copy(tbl_hbm.at[idx_v], rows_v, sem).wait()
        pltpu.sync_copy(rows_v, out_hbm.at[pl.ds(off, _CHUNK)])


_gather = functools.partial(
    pl.kernel,
    mesh=plsc.VectorSubcoreMesh(core_axis_name="c", subcore_axis_name="s"),
    out_type=jax.ShapeDtypeStruct((_N,), jnp.float32),
    scratch_types=[
        pltpu.VMEM((_CHUNK,), jnp.int32),
        pltpu.VMEM((_CHUNK,), jnp.float32),
        pltpu.SemaphoreType.DMA,
    ],
)(_gather_body)


def kernel(item_nos, user_params, item_table):
    idx = item_nos.reshape(-1).astype(jnp.int32)
    t = item_table.reshape(-1)
    t_pad = jnp.pad(t, (0, _PAD_N - _NUM_ITEMS)).reshape(_PAD_ROWS, 128)
    f = _sigmoid_table(user_params, t_pad).reshape(-1)
    out = _gather(f, idx)
    return out.reshape(_BATCH, _HIST)


# speedup vs baseline: 122.6818x; 122.6818x over previous
import functools

import jax
import jax.numpy as jnp
from jax import lax
from jax.experimental import pallas as pl
from jax.experimental.pallas import tpu as pltpu
from jax.experimental.pallas import tpu_sc as plsc

_NUM_ITEMS = 1000000
_BATCH = 16384
_HIST = 200
_N = _BATCH * _HIST

_PAD_ROWS = 8192
_PAD_N = _PAD_ROWS * 128

_NC, _NS = 2, 16
_NW = _NC * _NS
_PER_W = _N // _NW
_CHUNK = 51200
_NCHUNKS = _PER_W // _CHUNK


def _sigmoid_body(u_ref, x_ref, o_ref):
    o_ref[...] = jax.nn.sigmoid(u_ref[0] - x_ref[...])


def _sigmoid_table(user_params, table_pad):
    return pl.pallas_call(
        _sigmoid_body,
        out_shape=jax.ShapeDtypeStruct((_PAD_ROWS, 128), jnp.float32),
        in_specs=[
            pl.BlockSpec(memory_space=pltpu.SMEM),
            pl.BlockSpec(memory_space=pltpu.VMEM),
        ],
        out_specs=pl.BlockSpec(memory_space=pltpu.VMEM),
    )(user_params, table_pad)


def _gather_body(tbl_hbm, idx_hbm, out_hbm, idx_v, rows_v, sem):
    wid = lax.axis_index("s") * _NC + lax.axis_index("c")
    base = wid * _PER_W
    for k in range(_NCHUNKS):
        off = base + k * _CHUNK
        pltpu.sync_copy(idx_hbm.at[pl.ds(off, _CHUNK)], idx_v)
        pltpu.async_copy(tbl_hbm.at[idx_v], rows_v, sem).wait()
        pltpu.sync_copy(rows_v, out_hbm.at[pl.ds(off, _CHUNK)])


@functools.cache
def _make_gather():
    return pl.kernel(
        _gather_body,
        mesh=plsc.VectorSubcoreMesh(core_axis_name="c", subcore_axis_name="s"),
        out_type=jax.ShapeDtypeStruct((_N,), jnp.float32),
        scratch_types=[
            pltpu.VMEM((_CHUNK,), jnp.int32),
            pltpu.VMEM((_CHUNK,), jnp.float32),
            pltpu.SemaphoreType.DMA,
        ],
    )


def kernel(item_nos, user_params, item_table):
    idx = item_nos.reshape(-1).astype(jnp.int32)
    t = item_table.reshape(-1)
    t_pad = jnp.pad(t, (0, _PAD_N - _NUM_ITEMS)).reshape(_PAD_ROWS, 128)
    f = _sigmoid_table(user_params, t_pad).reshape(-1)
    out = _make_gather()(f, idx)
    return out.reshape(_BATCH, _HIST)


# speedup vs baseline: 177.7554x; 1.4489x over previous
import functools

import jax
import jax.numpy as jnp
from jax import lax
from jax.experimental import pallas as pl
from jax.experimental.pallas import tpu as pltpu
from jax.experimental.pallas import tpu_sc as plsc

_NUM_ITEMS = 1000000
_BATCH = 16384
_HIST = 200
_N = _BATCH * _HIST

_PAD_ROWS = 8192
_PAD_N = _PAD_ROWS * 128

_NC, _NS = 2, 16
_NW = _NC * _NS
_PER_W = _N // _NW
_CHUNK = 25600
_NCHUNKS = _PER_W // _CHUNK


def _sigmoid_body(u_ref, x_ref, o_ref):
    o_ref[...] = jax.nn.sigmoid(u_ref[0] - x_ref[...])


def _sigmoid_table(user_params, table_pad):
    return pl.pallas_call(
        _sigmoid_body,
        out_shape=jax.ShapeDtypeStruct((_PAD_ROWS, 128), jnp.float32),
        in_specs=[
            pl.BlockSpec(memory_space=pltpu.SMEM),
            pl.BlockSpec(memory_space=pltpu.VMEM),
        ],
        out_specs=pl.BlockSpec(memory_space=pltpu.VMEM),
    )(user_params, table_pad)


_TSLICE = _PAD_N // _NS


def _gather_body(tbl_hbm, idx_hbm, out_hbm, tbl_sh, idx_v, rows_v, sem):
    wid = lax.axis_index("s") * _NC + lax.axis_index("c")
    sid = lax.axis_index("s")
    toff = sid * _TSLICE
    pltpu.sync_copy(tbl_hbm.at[pl.ds(toff, _TSLICE)], tbl_sh.at[pl.ds(toff, _TSLICE)])
    plsc.subcore_barrier()
    base = wid * _PER_W
    for k in range(_NCHUNKS):
        off = base + k * _CHUNK
        pltpu.sync_copy(idx_hbm.at[pl.ds(off, _CHUNK)], idx_v)
        pltpu.async_copy(tbl_sh.at[idx_v], rows_v, sem).wait()
        pltpu.sync_copy(rows_v, out_hbm.at[pl.ds(off, _CHUNK)])


@functools.cache
def _make_gather():
    return pl.kernel(
        _gather_body,
        mesh=plsc.VectorSubcoreMesh(core_axis_name="c", subcore_axis_name="s"),
        out_type=jax.ShapeDtypeStruct((_N,), jnp.float32),
        scratch_types=[
            pltpu.VMEM_SHARED((_PAD_N,), jnp.float32),
            pltpu.VMEM((_CHUNK,), jnp.int32),
            pltpu.VMEM((_CHUNK,), jnp.float32),
            pltpu.SemaphoreType.DMA,
        ],
    )


def kernel(item_nos, user_params, item_table):
    idx = item_nos.reshape(-1).astype(jnp.int32)
    t = item_table.reshape(-1)
    t_pad = jnp.pad(t, (0, _PAD_N - _NUM_ITEMS)).reshape(_PAD_ROWS, 128)
    f = _sigmoid_table(user_params, t_pad).reshape(-1)
    out = _make_gather()(f, idx)
    return out.reshape(_BATCH, _HIST)
